# baseline (speedup 1.0000x reference)
import functools

import jax
import jax.numpy as jnp
from jax import lax
from jax.experimental import pallas as pl
from jax.experimental.pallas import tpu as pltpu
from jax.experimental.pallas import tpu_sc as plsc

_LANES = 16
_U_CHUNKS = 4


def _booleanize_sc(mask_flat):
    n = mask_flat.shape[0]
    info = plsc.get_sparse_core_info()
    nc, ns = info.num_cores, info.num_subcores
    nw = nc * ns
    per_w = n // nw
    assert per_w % _LANES == 0 and n % nw == 0

    mesh = plsc.VectorSubcoreMesh(core_axis_name="c", subcore_axis_name="s")

    @functools.partial(
        pl.kernel,
        mesh=mesh,
        out_type=jax.ShapeDtypeStruct((n,), jnp.int32),
        compiler_params=pltpu.CompilerParams(needs_layout_passes=False),
        scratch_types=[
            pltpu.VMEM((per_w,), jnp.float32),
            pltpu.VMEM((per_w,), jnp.int32),
        ],
    )
    def k(m_hbm, out_hbm, m_v, o_v):
        wid = lax.axis_index("s") * nc + lax.axis_index("c")
        base = wid * per_w
        pltpu.sync_copy(m_hbm.at[pl.ds(base, per_w)], m_v)
        for i in range(per_w // _LANES):
            sl = pl.ds(i * _LANES, _LANES)
            o_v[sl] = (m_v[sl] != 0.0).astype(jnp.int32)
        pltpu.sync_copy(o_v, out_hbm.at[pl.ds(base, per_w)])

    return k(mask_flat)


def _passthrough_tc(x, v, U):
    n_dma = 2 + _U_CHUNKS
    u_rows = U.shape[0] // _U_CHUNKS

    def body(x_in, v_in, u_in, x_out, v_out, u_out, *sems):
        copies = [
            pltpu.make_async_copy(x_in, x_out, sems[0]),
            pltpu.make_async_copy(v_in, v_out, sems[1]),
        ]
        for c in range(_U_CHUNKS):
            sl = pl.ds(c * u_rows, u_rows)
            copies.append(
                pltpu.make_async_copy(u_in.at[sl], u_out.at[sl], sems[2 + c])
            )
        for cp in copies:
            cp.start()
        for cp in copies:
            cp.wait()

    return pl.pallas_call(
        body,
        in_specs=[pl.BlockSpec(memory_space=pl.ANY)] * 3,
        out_specs=[pl.BlockSpec(memory_space=pl.ANY)] * 3,
        out_shape=[
            jax.ShapeDtypeStruct(x.shape, x.dtype),
            jax.ShapeDtypeStruct(v.shape, v.dtype),
            jax.ShapeDtypeStruct(U.shape, U.dtype),
        ],
        scratch_shapes=[pltpu.SemaphoreType.DMA] * n_dma,
    )(x, v, U)


def kernel(x, v, mask, U):
    mi = _booleanize_sc(mask.reshape(-1))
    ox, ov, oU = _passthrough_tc(x, v, U)
    mb = mi.astype(jnp.bool_).reshape(mask.shape)
    return (ox, ov, mb, oU)


# speedup vs baseline: 28.1230x; 28.1230x over previous
import functools

import jax
import jax.numpy as jnp
from jax import lax
from jax.experimental import pallas as pl
from jax.experimental.pallas import tpu as pltpu
from jax.experimental.pallas import tpu_sc as plsc

_LANES = 16
_U_CHUNKS = 4


def _booleanize_sc(mask_flat):
    n = mask_flat.shape[0]
    info = plsc.get_sparse_core_info()
    nc, ns = info.num_cores, info.num_subcores
    nw = nc * ns
    per_w = n // nw
    assert per_w % _LANES == 0 and n % nw == 0

    mesh = plsc.VectorSubcoreMesh(core_axis_name="c", subcore_axis_name="s")

    @functools.partial(
        pl.kernel,
        mesh=mesh,
        out_type=jax.ShapeDtypeStruct((n,), jnp.int32),
        compiler_params=pltpu.CompilerParams(needs_layout_passes=False),
        scratch_types=[
            pltpu.VMEM((per_w,), jnp.float32),
            pltpu.VMEM((per_w,), jnp.int32),
        ],
    )
    def k(m_hbm, out_hbm, m_v, o_v):
        wid = lax.axis_index("s") * nc + lax.axis_index("c")
        base = wid * per_w
        pltpu.sync_copy(m_hbm.at[pl.ds(base, per_w)], m_v)
        for i in range(per_w // _LANES):
            sl = pl.ds(i * _LANES, _LANES)
            o_v[sl] = (m_v[sl] != 0.0).astype(jnp.int32)
        pltpu.sync_copy(o_v, out_hbm.at[pl.ds(base, per_w)])

    return k(mask_flat)


def _copy_u_tc(U):
    B, H, L1, L2 = U.shape

    def body(u_in, u_out):
        u_out[...] = u_in[...]

    return pl.pallas_call(
        body,
        grid=(B, H),
        in_specs=[pl.BlockSpec((1, 1, L1, L2), lambda b, h: (b, h, 0, 0))],
        out_specs=pl.BlockSpec((1, 1, L1, L2), lambda b, h: (b, h, 0, 0)),
        out_shape=jax.ShapeDtypeStruct(U.shape, U.dtype),
        compiler_params=pltpu.CompilerParams(
            dimension_semantics=("parallel", "parallel"),
        ),
    )(U)


def kernel(x, v, mask, U):
    mi = _booleanize_sc(mask.reshape(-1))
    oU = _copy_u_tc(U)
    mb = mi.astype(jnp.bool_).reshape(mask.shape)
    return (x, v, mb, oU)


# speedup vs baseline: 44.8455x; 1.5946x over previous
import functools

import jax
import jax.numpy as jnp
from jax import lax
from jax.experimental import pallas as pl
from jax.experimental.pallas import tpu as pltpu
from jax.experimental.pallas import tpu_sc as plsc

_LANES = 16
_U_CHUNKS = 4


def _booleanize_sc(mask_flat):
    n = mask_flat.shape[0]
    info = plsc.get_sparse_core_info()
    nc, ns = info.num_cores, info.num_subcores
    nw = nc * ns
    per_w = n // nw
    assert per_w % _LANES == 0 and n % nw == 0

    mesh = plsc.VectorSubcoreMesh(core_axis_name="c", subcore_axis_name="s")

    @functools.partial(
        pl.kernel,
        mesh=mesh,
        out_type=jax.ShapeDtypeStruct((n,), jnp.int32),
        compiler_params=pltpu.CompilerParams(needs_layout_passes=False),
        scratch_types=[
            pltpu.VMEM((per_w,), jnp.float32),
            pltpu.VMEM((per_w,), jnp.int32),
        ],
    )
    def k(m_hbm, out_hbm, m_v, o_v):
        wid = lax.axis_index("s") * nc + lax.axis_index("c")
        base = wid * per_w
        pltpu.sync_copy(m_hbm.at[pl.ds(base, per_w)], m_v)
        for i in range(per_w // _LANES):
            sl = pl.ds(i * _LANES, _LANES)
            o_v[sl] = (m_v[sl] != 0.0).astype(jnp.int32)
        pltpu.sync_copy(o_v, out_hbm.at[pl.ds(base, per_w)])

    return k(mask_flat)


def _booleanize_tc(mask2d):
    def body(m_ref, o_ref):
        o_ref[...] = m_ref[...] != 0.0

    return pl.pallas_call(
        body,
        out_shape=jax.ShapeDtypeStruct(mask2d.shape, jnp.bool_),
    )(mask2d)


def kernel(x, v, mask, U):
    b, _, l = mask.shape
    mb = _booleanize_tc(mask.reshape(b, l)).reshape(mask.shape)
    return (x, v, mb, U)
